# Initial kernel scaffold; baseline (speedup 1.0000x reference)
import jax
import jax.numpy as jnp
from jax.experimental import pallas as pl


def kernel(x_cir, x_dis, cc_matrix, cc_edges, dd_matrix, dd_edges, W_cir1, b_cir1, W_cir2, b_cir2, W_dis1, b_dis1, W_dis2, b_dis2, W_cnn_cir, b_cnn_cir, W_cnn_dis, b_cnn_dis):
    raise NotImplementedError("write your pallas kernel here")


# speedup vs baseline: 11.1766x; 11.1766x over previous
import functools

import jax
import jax.numpy as jnp
from jax import lax
from jax.experimental import pallas as pl
from jax.experimental.pallas import tpu as pltpu
from jax.experimental.pallas import tpu_sc as plsc

_N_CIR = 585
_N_DIS = 88
_D = 128
_E_CC = 11700
_E_DD = 1760

_N_CIR_P = 640
_N_DIS_P = 128
_E_CC_P = 12288
_E_DD_P = 2048

_NC = 2
_NS = 16
_NW = _NC * _NS
_CC_ROWS = _N_CIR_P // _NW
_DD_ROWS = _N_DIS_P // _NW


def _sc_count_matrices(ecc, edd, zeros):
    mesh = plsc.VectorSubcoreMesh(core_axis_name="c", subcore_axis_name="s")

    @functools.partial(
        pl.kernel,
        out_type=(
            jax.ShapeDtypeStruct((_N_CIR_P * _N_CIR_P,), jnp.float32),
            jax.ShapeDtypeStruct((_N_DIS_P * _N_DIS_P,), jnp.float32),
        ),
        mesh=mesh,
        compiler_params=pltpu.CompilerParams(needs_layout_passes=False),
        scratch_types=[
            pltpu.VMEM((2, _E_CC_P), jnp.int32),
            pltpu.VMEM((2, _E_DD_P), jnp.int32),
            pltpu.VMEM((_CC_ROWS * _N_CIR_P,), jnp.float32),
            pltpu.VMEM((_DD_ROWS * _N_DIS_P,), jnp.float32),
        ],
    )
    def k(ecc_hbm, edd_hbm, zeros_hbm, outc_hbm, outd_hbm,
          ecc_v, edd_v, cntc_v, cntd_v):
        wid = lax.axis_index("s") * _NC + lax.axis_index("c")
        pltpu.sync_copy(ecc_hbm, ecc_v)
        pltpu.sync_copy(edd_hbm, edd_v)
        pltpu.sync_copy(zeros_hbm, cntc_v)
        pltpu.sync_copy(zeros_hbm.at[pl.ds(0, _DD_ROWS * _N_DIS_P)], cntd_v)

        lane = lax.iota(jnp.int32, 16)
        ones = jnp.full((16,), 1.0, jnp.float32)

        def edge_scan(ev, cnt_v, n_vec, lo, hi, npad):
            def body(i, carry):
                base = i * 16
                r = ev[0, pl.ds(base, 16)]
                c = ev[1, pl.ds(base, 16)]
                m = (c >= lo) & (c < hi)
                li = (c - lo) * npad + r
                for j in range(16):
                    plsc.addupdate_scatter(
                        cnt_v, [li], ones, mask=m & (lane == j))
                return carry

            lax.fori_loop(0, n_vec, body, 0)

        lo_c = wid * _CC_ROWS
        edge_scan(ecc_v, cntc_v, _E_CC_P // 16, lo_c, lo_c + _CC_ROWS, _N_CIR_P)
        lo_d = wid * _DD_ROWS
        edge_scan(edd_v, cntd_v, _E_DD_P // 16, lo_d, lo_d + _DD_ROWS, _N_DIS_P)

        pltpu.sync_copy(
            cntc_v,
            outc_hbm.at[pl.ds(wid * _CC_ROWS * _N_CIR_P, _CC_ROWS * _N_CIR_P)])
        pltpu.sync_copy(
            cntd_v,
            outd_hbm.at[pl.ds(wid * _DD_ROWS * _N_DIS_P, _DD_ROWS * _N_DIS_P)])

    return k(ecc, edd, zeros)


def _tc_dense_body(cntc_ref, mct_ref, xc_ref, wc1_ref, bc1_ref, wc2_ref,
                   bc2_ref, uc0_ref, uc1_ref, bcc_ref,
                   cntd_ref, mdt_ref, xd_ref, wd1_ref, bd1_ref, wd2_ref,
                   bd2_ref, ud0_ref, ud1_ref, bdc_ref,
                   score_ref, cir_ref, dis_ref):
    f32 = jnp.float32

    def side(cnt, mt, x, w1, b1, w2, b2, u0, u1, bc, n):
        B = cnt * mt
        ones = jnp.ones((n, 1), f32)
        deg = 1.0 + jnp.dot(B, ones, preferred_element_type=f32)
        dinv = lax.rsqrt(deg)

        def gcn(xin, W, b):
            h = jnp.dot(xin, W, preferred_element_type=f32)
            t = dinv * h
            u = jnp.dot(B, t, preferred_element_type=f32)
            return jnp.maximum(dinv * u + (dinv * dinv) * h + b, 0.0)

        f1 = gcn(x, w1, b1)
        f2 = gcn(f1, w2, b2)
        return (jnp.dot(f1, u0, preferred_element_type=f32)
                + jnp.dot(f2, u1, preferred_element_type=f32) + bc)

    cir = side(cntc_ref[...], mct_ref[...], xc_ref[...], wc1_ref[...],
               bc1_ref[...], wc2_ref[...], bc2_ref[...], uc0_ref[...],
               uc1_ref[...], bcc_ref[...], _N_CIR_P)
    dis = side(cntd_ref[...], mdt_ref[...], xd_ref[...], wd1_ref[...],
               bd1_ref[...], wd2_ref[...], bd2_ref[...], ud0_ref[...],
               ud1_ref[...], bdc_ref[...], _N_DIS_P)
    score_ref[...] = lax.dot_general(
        cir, dis, (((1,), (1,)), ((), ())), preferred_element_type=f32)
    cir_ref[...] = cir
    dis_ref[...] = dis


def kernel(x_cir, x_dis, cc_matrix, cc_edges, dd_matrix, dd_edges,
           W_cir1, b_cir1, W_cir2, b_cir2, W_dis1, b_dis1, W_dis2, b_dis2,
           W_cnn_cir, b_cnn_cir, W_cnn_dis, b_cnn_dis):
    f32 = jnp.float32

    pad_cc = jnp.broadcast_to(
        jnp.array([[0], [_N_CIR_P - 1]], jnp.int32), (2, _E_CC_P - _E_CC))
    pad_dd = jnp.broadcast_to(
        jnp.array([[0], [_N_DIS_P - 1]], jnp.int32), (2, _E_DD_P - _E_DD))
    ecc = jnp.concatenate([cc_edges.astype(jnp.int32), pad_cc], axis=1)
    edd = jnp.concatenate([dd_edges.astype(jnp.int32), pad_dd], axis=1)
    zeros = jnp.zeros((_CC_ROWS * _N_CIR_P,), f32)

    cntc_flat, cntd_flat = _sc_count_matrices(ecc, edd, zeros)
    cntc = cntc_flat.reshape(_N_CIR_P, _N_CIR_P)
    cntd = cntd_flat.reshape(_N_DIS_P, _N_DIS_P)

    pc = _N_CIR_P - _N_CIR
    pd = _N_DIS_P - _N_DIS
    mct = jnp.pad(cc_matrix.T, ((0, pc), (0, pc)))
    mdt = jnp.pad(dd_matrix.T, ((0, pd), (0, pd)))
    xc = jnp.pad(x_cir, ((0, pc), (0, 0)))
    xd = jnp.pad(x_dis, ((0, pd), (0, 0)))
    uc0 = W_cnn_cir[:, 0, :].T
    uc1 = W_cnn_cir[:, 1, :].T
    ud0 = W_cnn_dis[:, 0, :].T
    ud1 = W_cnn_dis[:, 1, :].T

    score, cir, dis = pl.pallas_call(
        _tc_dense_body,
        out_shape=(
            jax.ShapeDtypeStruct((_N_CIR_P, _N_DIS_P), f32),
            jax.ShapeDtypeStruct((_N_CIR_P, 256), f32),
            jax.ShapeDtypeStruct((_N_DIS_P, 256), f32),
        ),
    )(cntc, mct, xc, W_cir1, b_cir1.reshape(1, _D), W_cir2,
      b_cir2.reshape(1, _D), uc0, uc1, b_cnn_cir.reshape(1, 256),
      cntd, mdt, xd, W_dis1, b_dis1.reshape(1, _D), W_dis2,
      b_dis2.reshape(1, _D), ud0, ud1, b_cnn_dis.reshape(1, 256))

    return (score[:_N_CIR, :_N_DIS], cir[:_N_CIR], dis[:_N_DIS])
